# initial kernel scaffold (unmeasured)
import jax
import jax.numpy as jnp
from jax import lax
from jax.experimental import pallas as pl
from jax.experimental.pallas import tpu as pltpu

N_DEV = 4
B_SH = 64
B = N_DEV * B_SH
D = 512
N_HOPS = 18
N_SLOTS = 30


def kernel(x, Win0, Wout0, Win1, Wout1, Win2, Wout2):
    def body(x_ref, win0_ref, wout0_ref, win1_ref, wout1_ref, win2_ref,
             wout2_ref, out_ref, xfull_ref, part_ref, comm_ref,
             send_sems, recv_sems):
        my = lax.axis_index("i")
        left = lax.rem(my + N_DEV - 1, N_DEV)
        right = lax.rem(my + 1, N_DEV)

        barrier_sem = pltpu.get_barrier_semaphore()
        for nbr in (left, right):
            pl.semaphore_signal(
                barrier_sem, inc=1,
                device_id=(nbr,), device_id_type=pl.DeviceIdType.MESH,
            )
        pl.semaphore_wait(barrier_sem, 2)

        hop_counter = [0]

        def ring_hop(src_slot, dst_slot):
            h = hop_counter[0]
            hop_counter[0] += 1
            rdma = pltpu.make_async_remote_copy(
                src_ref=comm_ref.at[src_slot],
                dst_ref=comm_ref.at[dst_slot],
                send_sem=send_sems.at[h],
                recv_sem=recv_sems.at[h],
                device_id=(right,),
                device_id_type=pl.DeviceIdType.MESH,
            )
            rdma.start()
            rdma.wait()

        def rows(c):
            return pl.ds(c * B_SH, B_SH)

        xfull_ref[rows(my), :] = x_ref[:, :]
        comm_ref[0] = x_ref[:, :]
        for h in range(N_DEV - 1):
            ring_hop(h, h + 1)
            origin = lax.rem(my + 2 * N_DEV - h - 2, N_DEV)
            xfull_ref[rows(origin), :] = comm_ref[h + 1]

        slot = N_DEV

        layers = ((win0_ref, wout0_ref), (win1_ref, wout1_ref),
                  (win2_ref, wout2_ref))
        for layer, (win_ref, wout_ref) in enumerate(layers):
            hmat = jnp.maximum(
                jnp.dot(xfull_ref[:, :], win_ref[:, :],
                        preferred_element_type=jnp.float32),
                0.0,
            )
            part_ref[:, :] = jnp.dot(
                hmat, wout_ref[:, :], preferred_element_type=jnp.float32)

            s0 = slot
            c = lax.rem(my + N_DEV - 1, N_DEV)
            comm_ref[s0] = part_ref[rows(c), :]
            acc = None
            for s in range(N_DEV - 1):
                ring_hop(s0 + 2 * s, s0 + 2 * s + 1)
                c = lax.rem(my + 2 * N_DEV - s - 2, N_DEV)
                acc = comm_ref[s0 + 2 * s + 1] + part_ref[rows(c), :]
                if s < N_DEV - 2:
                    comm_ref[s0 + 2 * s + 2] = acc
            slot = s0 + 2 * (N_DEV - 1)

            if layer < len(layers) - 1:
                t0 = slot
                xfull_ref[rows(my), :] = acc
                comm_ref[t0] = acc
                for h in range(N_DEV - 1):
                    ring_hop(t0 + h, t0 + h + 1)
                    origin = lax.rem(my + 2 * N_DEV - h - 2, N_DEV)
                    xfull_ref[rows(origin), :] = comm_ref[t0 + h + 1]
                slot = t0 + N_DEV
            else:
                out_ref[:, :] = acc

    return pl.pallas_call(
        body,
        out_shape=jax.ShapeDtypeStruct((B_SH, D), jnp.float32),
        in_specs=[pl.BlockSpec(memory_space=pltpu.VMEM)] * 7,
        out_specs=pl.BlockSpec(memory_space=pltpu.VMEM),
        scratch_shapes=[
            pltpu.VMEM((B, D), jnp.float32),
            pltpu.VMEM((B, D), jnp.float32),
            pltpu.VMEM((N_SLOTS, B_SH, D), jnp.float32),
            pltpu.SemaphoreType.DMA((N_HOPS,)),
            pltpu.SemaphoreType.DMA((N_HOPS,)),
        ],
        compiler_params=pltpu.CompilerParams(collective_id=0),
    )(x, Win0, Wout0, Win1, Wout1, Win2, Wout2)


# baseline (device time: 71853 ns/iter reference)
import jax
import jax.numpy as jnp
from jax import lax
from jax.experimental import pallas as pl
from jax.experimental.pallas import tpu as pltpu

N_DEV = 4
B_SH = 64
B = N_DEV * B_SH
D = 512
N_HOPS = 18
N_SLOTS = 30


def kernel(x, Win0, Wout0, Win1, Wout1, Win2, Wout2):
    def body(x_ref, win0_ref, wout0_ref, win1_ref, wout1_ref, win2_ref,
             wout2_ref, out_ref, xfull_ref, part_ref, comm_ref,
             send_sems, recv_sems):
        my = lax.axis_index("i")
        left = lax.rem(my + N_DEV - 1, N_DEV)
        right = lax.rem(my + 1, N_DEV)

        barrier_sem = pltpu.get_barrier_semaphore()
        for nbr in (left, right):
            pl.semaphore_signal(
                barrier_sem, inc=1,
                device_id=(nbr,), device_id_type=pl.DeviceIdType.MESH,
            )
        pl.semaphore_wait(barrier_sem, 2)

        hop_counter = [0]

        def ring_hop(src_slot, dst_slot):
            h = hop_counter[0]
            hop_counter[0] += 1
            rdma = pltpu.make_async_remote_copy(
                src_ref=comm_ref.at[src_slot],
                dst_ref=comm_ref.at[dst_slot],
                send_sem=send_sems.at[h],
                recv_sem=recv_sems.at[h],
                device_id=(right,),
                device_id_type=pl.DeviceIdType.MESH,
            )
            rdma.start()
            rdma.wait()

        def rows(c):
            return pl.ds(c * B_SH, B_SH)

        xfull_ref[rows(my), :] = x_ref[:, :]
        comm_ref[0] = x_ref[:, :]
        for h in range(N_DEV - 1):
            ring_hop(h, h + 1)
            origin = lax.rem(my + 2 * N_DEV - h - 1, N_DEV)
            xfull_ref[rows(origin), :] = comm_ref[h + 1]

        slot = N_DEV

        layers = ((win0_ref, wout0_ref), (win1_ref, wout1_ref),
                  (win2_ref, wout2_ref))
        for layer, (win_ref, wout_ref) in enumerate(layers):
            hmat = jnp.maximum(
                jnp.dot(xfull_ref[:, :], win_ref[:, :],
                        preferred_element_type=jnp.float32),
                0.0,
            )
            part_ref[:, :] = jnp.dot(
                hmat, wout_ref[:, :], preferred_element_type=jnp.float32)

            s0 = slot
            c = lax.rem(my + N_DEV - 1, N_DEV)
            comm_ref[s0] = part_ref[rows(c), :]
            acc = None
            for s in range(N_DEV - 1):
                ring_hop(s0 + 2 * s, s0 + 2 * s + 1)
                c = lax.rem(my + 2 * N_DEV - s - 2, N_DEV)
                acc = comm_ref[s0 + 2 * s + 1] + part_ref[rows(c), :]
                if s < N_DEV - 2:
                    comm_ref[s0 + 2 * s + 2] = acc
            slot = s0 + 2 * (N_DEV - 1)

            if layer < len(layers) - 1:
                t0 = slot
                xfull_ref[rows(my), :] = acc
                comm_ref[t0] = acc
                for h in range(N_DEV - 1):
                    ring_hop(t0 + h, t0 + h + 1)
                    origin = lax.rem(my + 2 * N_DEV - h - 1, N_DEV)
                    xfull_ref[rows(origin), :] = comm_ref[t0 + h + 1]
                slot = t0 + N_DEV
            else:
                out_ref[:, :] = acc

    return pl.pallas_call(
        body,
        out_shape=jax.ShapeDtypeStruct((B_SH, D), jnp.float32),
        in_specs=[pl.BlockSpec(memory_space=pltpu.VMEM)] * 7,
        out_specs=pl.BlockSpec(memory_space=pltpu.VMEM),
        scratch_shapes=[
            pltpu.VMEM((B, D), jnp.float32),
            pltpu.VMEM((B, D), jnp.float32),
            pltpu.VMEM((N_SLOTS, B_SH, D), jnp.float32),
            pltpu.SemaphoreType.DMA((N_HOPS,)),
            pltpu.SemaphoreType.DMA((N_HOPS,)),
        ],
        compiler_params=pltpu.CompilerParams(collective_id=0),
    )(x, Win0, Wout0, Win1, Wout1, Win2, Wout2)


# device time: 45665 ns/iter; 1.5735x vs baseline; 1.5735x over previous
import jax
import jax.numpy as jnp
from jax import lax
from jax.experimental import pallas as pl
from jax.experimental.pallas import tpu as pltpu

N_DEV = 4
B_SH = 64
B = N_DEV * B_SH
HALF = B // 2
D = 512
N_RDMA = 14


def kernel(x, Win0, Wout0, Win1, Wout1, Win2, Wout2):
    def body(x_ref, win0_ref, wout0_ref, win1_ref, wout1_ref, win2_ref,
             wout2_ref, out_ref, xfull_ref, part_ref, ag_ref, arh_ref,
             rs_ref, send_sems, recv_sems):
        my = lax.axis_index("i")
        y_p = my ^ 1
        x_p = 3 - my
        d_p = (3 - my) ^ 1

        barrier_sem = pltpu.get_barrier_semaphore()
        for nbr in (y_p, x_p):
            pl.semaphore_signal(
                barrier_sem, inc=1,
                device_id=(nbr,), device_id_type=pl.DeviceIdType.MESH,
            )
        pl.semaphore_wait(barrier_sem, 2)

        sem_counter = [0]

        def rdma(src, dst, target):
            i = sem_counter[0]
            sem_counter[0] += 1
            return pltpu.make_async_remote_copy(
                src_ref=src, dst_ref=dst,
                send_sem=send_sems.at[i], recv_sem=recv_sems.at[i],
                device_id=(target,), device_id_type=pl.DeviceIdType.MESH,
            )

        def rows(c):
            return pl.ds(c * B_SH, B_SH)

        r0 = rdma(x_ref, ag_ref.at[0], y_p)
        r1 = rdma(x_ref, ag_ref.at[1], x_p)
        r0.start()
        r1.start()
        r0.wait()
        r1.wait()
        xfull_ref[rows(my), :] = x_ref[:, :]
        xfull_ref[rows(y_p), :] = ag_ref[0]
        xfull_ref[rows(x_p), :] = ag_ref[1]
        r2 = rdma(ag_ref.at[0], ag_ref.at[2], x_p)
        r2.start()
        r2.wait()
        xfull_ref[rows(d_p), :] = ag_ref[2]

        layers = ((win0_ref, wout0_ref), (win1_ref, wout1_ref),
                  (win2_ref, wout2_ref))
        for layer, (win_ref, wout_ref) in enumerate(layers):
            hmat = jnp.maximum(
                jnp.dot(xfull_ref[:, :], win_ref[:, :],
                        preferred_element_type=jnp.float32),
                0.0,
            )
            part_ref[:, :] = jnp.dot(
                hmat, wout_ref[:, :], preferred_element_type=jnp.float32)

            if layer < len(layers) - 1:
                a0 = 4 * layer
                ra = rdma(part_ref.at[pl.ds(0, HALF)], arh_ref.at[a0], y_p)
                rb = rdma(part_ref.at[pl.ds(HALF, HALF)], arh_ref.at[a0 + 1],
                          x_p)
                ra.start()
                rb.start()
                ra.wait()
                rb.wait()
                xfull_ref[pl.ds(0, HALF), :] = (
                    part_ref[pl.ds(0, HALF), :] + arh_ref[a0])
                xfull_ref[pl.ds(HALF, HALF), :] = (
                    part_ref[pl.ds(HALF, HALF), :] + arh_ref[a0 + 1])
                ra2 = rdma(xfull_ref.at[pl.ds(0, HALF)], arh_ref.at[a0 + 2],
                           x_p)
                rb2 = rdma(xfull_ref.at[pl.ds(HALF, HALF)],
                           arh_ref.at[a0 + 3], y_p)
                ra2.start()
                rb2.start()
                ra2.wait()
                rb2.wait()
                xfull_ref[pl.ds(0, HALF), :] = (
                    xfull_ref[pl.ds(0, HALF), :] + arh_ref[a0 + 2])
                xfull_ref[pl.ds(HALF, HALF), :] = (
                    xfull_ref[pl.ds(HALF, HALF), :] + arh_ref[a0 + 3])
            else:
                rq0 = rdma(part_ref.at[rows(y_p)], rs_ref.at[0], y_p)
                rq1 = rdma(part_ref.at[rows(x_p)], rs_ref.at[1], x_p)
                rq2 = rdma(part_ref.at[rows(d_p)], rs_ref.at[2], d_p)
                rq0.start()
                rq1.start()
                rq2.start()
                rq0.wait()
                rq1.wait()
                rq2.wait()
                out_ref[:, :] = (part_ref[rows(my), :] + rs_ref[0]
                                 + rs_ref[1] + rs_ref[2])

    return pl.pallas_call(
        body,
        out_shape=jax.ShapeDtypeStruct((B_SH, D), jnp.float32),
        in_specs=[pl.BlockSpec(memory_space=pltpu.VMEM)] * 7,
        out_specs=pl.BlockSpec(memory_space=pltpu.VMEM),
        scratch_shapes=[
            pltpu.VMEM((B, D), jnp.float32),
            pltpu.VMEM((B, D), jnp.float32),
            pltpu.VMEM((3, B_SH, D), jnp.float32),
            pltpu.VMEM((8, HALF, D), jnp.float32),
            pltpu.VMEM((3, B_SH, D), jnp.float32),
            pltpu.SemaphoreType.DMA((N_RDMA,)),
            pltpu.SemaphoreType.DMA((N_RDMA,)),
        ],
        compiler_params=pltpu.CompilerParams(collective_id=0),
    )(x, Win0, Wout0, Win1, Wout1, Win2, Wout2)
